# Initial kernel scaffold; baseline (speedup 1.0000x reference)
import functools

import jax
import jax.numpy as jnp
from jax import lax
from jax.experimental import pallas as pl
from jax.experimental.pallas import tpu as pltpu
from jax.experimental.pallas import tpu_sc as plsc

N = 10000
NPAD = 10240
D = 128
H = 128
C = 40
G = 64
E_TOT = 320000 + N
CHUNK = 128
NWORK = 32
NCH = -(-E_TOT // (CHUNK * NWORK))
E_PAD = NCH * CHUNK * NWORK
ROWS = NPAD // 16

_mesh = plsc.VectorSubcoreMesh(core_axis_name="c", subcore_axis_name="s")


@functools.partial(
    pl.kernel,
    out_type=jax.ShapeDtypeStruct((2, NPAD, 16), jnp.float32),
    mesh=_mesh,
    scratch_types=[
        pltpu.VMEM((CHUNK,), jnp.int32),
        pltpu.VMEM((CHUNK, 16), jnp.float32),
        pltpu.VMEM_SHARED((NPAD, 16), jnp.float32),
    ],
)
def _sc_degree(dst_hbm, ones_hbm, z16_hbm, out_hbm, dst_v, ones_v, acc):
    c = lax.axis_index("c")
    s = lax.axis_index("s")
    pltpu.sync_copy(z16_hbm, acc.at[pl.ds(s * ROWS, ROWS)])
    pltpu.sync_copy(ones_hbm, ones_v)
    plsc.subcore_barrier()
    base = (c * 16 + s) * NCH * CHUNK

    def body(g, carry):
        pltpu.sync_copy(dst_hbm.at[pl.ds(base + g * CHUNK, CHUNK)], dst_v)
        pltpu.sync_copy(ones_v, acc.at[dst_v], add=True)
        return carry

    lax.fori_loop(0, NCH, body, 0)
    plsc.subcore_barrier()
    pltpu.sync_copy(acc.at[pl.ds(s * ROWS, ROWS)],
                    out_hbm.at[c, pl.ds(s * ROWS, ROWS)])


@functools.partial(
    pl.kernel,
    out_type=jax.ShapeDtypeStruct((2, NPAD, H), jnp.float32),
    mesh=_mesh,
    scratch_types=[
        pltpu.VMEM((CHUNK,), jnp.int32),
        pltpu.VMEM((CHUNK,), jnp.int32),
        pltpu.VMEM((CHUNK, H), jnp.float32),
        pltpu.VMEM_SHARED((NPAD, H), jnp.float32),
        pltpu.SemaphoreType.DMA,
    ],
)
def _sc_gather_scatter(y_hbm, src_hbm, dst_hbm, z_hbm, out_hbm,
                       src_v, dst_v, rows_v, acc, sem):
    c = lax.axis_index("c")
    s = lax.axis_index("s")
    pltpu.sync_copy(z_hbm, acc.at[pl.ds(s * ROWS, ROWS)])
    plsc.subcore_barrier()
    base = (c * 16 + s) * NCH * CHUNK

    def body(g, carry):
        o = base + g * CHUNK
        pltpu.sync_copy(src_hbm.at[pl.ds(o, CHUNK)], src_v)
        pltpu.sync_copy(dst_hbm.at[pl.ds(o, CHUNK)], dst_v)
        pltpu.async_copy(y_hbm.at[src_v], rows_v, sem).wait()
        pltpu.sync_copy(rows_v, acc.at[dst_v], add=True)
        return carry

    lax.fori_loop(0, NCH, body, 0)
    plsc.subcore_barrier()
    pltpu.sync_copy(acc.at[pl.ds(s * ROWS, ROWS)],
                    out_hbm.at[c, pl.ds(s * ROWS, ROWS)])


def _valid_mask():
    return lax.broadcasted_iota(jnp.int32, (NPAD, 1), 0) < N


def _tc_prep(x_ref, w1_ref, deg_ref, y_ref, dinv_ref):
    deg16 = deg_ref[0] + deg_ref[1]
    deg = deg16[:, 0:1]
    dinv = jnp.where(deg > 0.0, lax.rsqrt(deg), 0.0)
    xw = jnp.dot(x_ref[...], w1_ref[...], preferred_element_type=jnp.float32)
    y_ref[...] = xw * dinv
    dinv_ref[...] = dinv


def _bn_relu(p0, p1, dinv, b, g, be):
    z = (p0 + p1) * dinv + b
    mask = _valid_mask()
    zm = jnp.where(mask, z, 0.0)
    m = jnp.sum(zm, axis=0, keepdims=True) * (1.0 / N)
    dz = jnp.where(mask, z - m, 0.0)
    v = jnp.sum(dz * dz, axis=0, keepdims=True) * (1.0 / N)
    return jnp.maximum(g * (z - m) * lax.rsqrt(v + 1e-5) + be, 0.0)


def _tc_mid(p_ref, dinv_ref, b_ref, g_ref, be_ref, wn_ref, y_ref):
    dinv = dinv_ref[...]
    h = _bn_relu(p_ref[0], p_ref[1], dinv, b_ref[...], g_ref[...], be_ref[...])
    y_ref[...] = jnp.dot(h, wn_ref[...], preferred_element_type=jnp.float32) * dinv


def _tc_final(p_ref, dinv_ref, b_ref, g_ref, be_ref, batch_ref, wl_ref,
              bl_ref, out_ref):
    h = _bn_relu(p_ref[0], p_ref[1], dinv_ref[...], b_ref[...], g_ref[...],
                 be_ref[...])
    h = jnp.where(_valid_mask(), h, 0.0)
    gids = lax.broadcasted_iota(jnp.int32, (NPAD, G), 1)
    oh = (batch_ref[...].reshape(NPAD, 1) == gids).astype(jnp.float32)
    seg = lax.dot_general(oh, h, (((0,), (0,)), ((), ())),
                          preferred_element_type=jnp.float32)
    cnt = jnp.sum(oh, axis=0, keepdims=True)
    pooled = seg / jnp.clip(cnt, 1.0, None).reshape(G, 1)
    out_ref[...] = (jnp.dot(pooled, wl_ref[...],
                            preferred_element_type=jnp.float32) + bl_ref[...])


def kernel(x, edge_index, batch, W1, b1, g1, be1, Wc, bc, gc, bec, Wl, bl):
    loop = jnp.arange(N, dtype=jnp.int32)
    src = jnp.concatenate([edge_index[0].astype(jnp.int32), loop,
                           jnp.zeros((E_PAD - E_TOT,), jnp.int32)])
    dst = jnp.concatenate([edge_index[1].astype(jnp.int32), loop,
                           jnp.full((E_PAD - E_TOT,), N, jnp.int32)])
    x_pad = jnp.zeros((NPAD, D), jnp.float32).at[:N].set(x)
    batch_pad = jnp.concatenate([batch.astype(jnp.int32),
                                 jnp.full((NPAD - N,), G, jnp.int32)])
    z_rows = jnp.zeros((ROWS, H), jnp.float32)
    z16 = jnp.zeros((ROWS, 16), jnp.float32)
    ones16 = jnp.ones((CHUNK, 16), jnp.float32)

    deg16 = _sc_degree(dst, ones16, z16)

    y, dinv = pl.pallas_call(
        _tc_prep,
        out_shape=(jax.ShapeDtypeStruct((NPAD, H), jnp.float32),
                   jax.ShapeDtypeStruct((NPAD, 1), jnp.float32)),
    )(x_pad, W1, deg16)

    biases = [b1, bc[0], bc[1]]
    gammas = [g1, gc[0], gc[1]]
    betas = [be1, bec[0], bec[1]]

    for layer in range(3):
        p = _sc_gather_scatter(y, src, dst, z_rows)
        b_ = biases[layer].reshape(1, H)
        g_ = gammas[layer].reshape(1, H)
        be_ = betas[layer].reshape(1, H)
        if layer < 2:
            y = pl.pallas_call(
                _tc_mid,
                out_shape=jax.ShapeDtypeStruct((NPAD, H), jnp.float32),
            )(p, dinv, b_, g_, be_, Wc[layer])
        else:
            out = pl.pallas_call(
                _tc_final,
                out_shape=jax.ShapeDtypeStruct((G, C), jnp.float32),
            )(p, dinv, b_, g_, be_, batch_pad, Wl, bl.reshape(1, C))
    return out


# speedup vs baseline: 10.9627x; 10.9627x over previous
import functools

import jax
import jax.numpy as jnp
from jax import lax
from jax.experimental import pallas as pl
from jax.experimental.pallas import tpu as pltpu
from jax.experimental.pallas import tpu_sc as plsc

N = 10000
NPAD = 10240
D = 128
H = 128
C = 40
G = 64
E_TOT = 320000 + N
CHUNK = 128
NWORK = 32
NCH = -(-E_TOT // (CHUNK * NWORK))
E_PAD = NCH * CHUNK * NWORK
ROWS = NPAD // 16

_mesh = plsc.VectorSubcoreMesh(core_axis_name="c", subcore_axis_name="s")


@functools.partial(
    pl.kernel,
    out_type=jax.ShapeDtypeStruct((2, NPAD, H), jnp.float32),
    mesh=_mesh,
    scratch_types=[
        pltpu.VMEM((CHUNK,), jnp.int32),
        pltpu.VMEM((CHUNK, H), jnp.float32),
        pltpu.VMEM_SHARED((NPAD, H), jnp.float32),
    ],
)
def _sc_degree(dst_hbm, ones_hbm, z16_hbm, out_hbm, dst_v, ones_v, acc):
    c = lax.axis_index("c")
    s = lax.axis_index("s")
    pltpu.sync_copy(z16_hbm, acc.at[pl.ds(s * ROWS, ROWS)])
    pltpu.sync_copy(ones_hbm, ones_v)
    plsc.subcore_barrier()
    base = (c * 16 + s) * NCH * CHUNK

    def body(g, carry):
        pltpu.sync_copy(dst_hbm.at[pl.ds(base + g * CHUNK, CHUNK)], dst_v)
        pltpu.sync_copy(ones_v, acc.at[dst_v], add=True)
        return carry

    lax.fori_loop(0, NCH, body, 0)
    plsc.subcore_barrier()
    pltpu.sync_copy(acc.at[pl.ds(s * ROWS, ROWS)],
                    out_hbm.at[c, pl.ds(s * ROWS, ROWS)])


@functools.partial(
    pl.kernel,
    out_type=jax.ShapeDtypeStruct((2, NPAD, H), jnp.float32),
    mesh=_mesh,
    scratch_types=[
        pltpu.VMEM((CHUNK,), jnp.int32),
        pltpu.VMEM((CHUNK,), jnp.int32),
        pltpu.VMEM((CHUNK, H), jnp.float32),
        pltpu.VMEM_SHARED((NPAD, H), jnp.float32),
        pltpu.SemaphoreType.DMA,
    ],
)
def _sc_gather_scatter(y_hbm, src_hbm, dst_hbm, z_hbm, out_hbm,
                       src_v, dst_v, rows_v, acc, sem):
    c = lax.axis_index("c")
    s = lax.axis_index("s")
    pltpu.sync_copy(z_hbm, acc.at[pl.ds(s * ROWS, ROWS)])
    plsc.subcore_barrier()
    base = (c * 16 + s) * NCH * CHUNK

    def body(g, carry):
        o = base + g * CHUNK
        pltpu.sync_copy(src_hbm.at[pl.ds(o, CHUNK)], src_v)
        pltpu.sync_copy(dst_hbm.at[pl.ds(o, CHUNK)], dst_v)
        pltpu.async_copy(y_hbm.at[src_v], rows_v, sem).wait()
        pltpu.sync_copy(rows_v, acc.at[dst_v], add=True)
        return carry

    lax.fori_loop(0, NCH, body, 0)
    plsc.subcore_barrier()
    pltpu.sync_copy(acc.at[pl.ds(s * ROWS, ROWS)],
                    out_hbm.at[c, pl.ds(s * ROWS, ROWS)])


def _valid_mask():
    return lax.broadcasted_iota(jnp.int32, (NPAD, 1), 0) < N


def _tc_prep(x_ref, w1_ref, deg_ref, y_ref, dinv_ref):
    deg16 = deg_ref[0] + deg_ref[1]
    deg = deg16[:, 0:1]
    dinv = jnp.where(deg > 0.0, lax.rsqrt(deg), 0.0)
    xw = jnp.dot(x_ref[...], w1_ref[...], preferred_element_type=jnp.float32)
    y_ref[...] = xw * dinv
    dinv_ref[...] = dinv


def _bn_relu(p0, p1, dinv, b, g, be):
    z = (p0 + p1) * dinv + b
    mask = _valid_mask()
    zm = jnp.where(mask, z, 0.0)
    m = jnp.sum(zm, axis=0, keepdims=True) * (1.0 / N)
    dz = jnp.where(mask, z - m, 0.0)
    v = jnp.sum(dz * dz, axis=0, keepdims=True) * (1.0 / N)
    return jnp.maximum(g * (z - m) * lax.rsqrt(v + 1e-5) + be, 0.0)


def _tc_mid(p_ref, dinv_ref, b_ref, g_ref, be_ref, wn_ref, y_ref):
    dinv = dinv_ref[...]
    h = _bn_relu(p_ref[0], p_ref[1], dinv, b_ref[...], g_ref[...], be_ref[...])
    y_ref[...] = jnp.dot(h, wn_ref[...], preferred_element_type=jnp.float32) * dinv


def _tc_final(p_ref, dinv_ref, b_ref, g_ref, be_ref, batch_ref, wl_ref,
              bl_ref, out_ref):
    h = _bn_relu(p_ref[0], p_ref[1], dinv_ref[...], b_ref[...], g_ref[...],
                 be_ref[...])
    h = jnp.where(_valid_mask(), h, 0.0)
    gids = lax.broadcasted_iota(jnp.int32, (NPAD, G), 1)
    oh = (batch_ref[...].reshape(NPAD, 1) == gids).astype(jnp.float32)
    seg = lax.dot_general(oh, h, (((0,), (0,)), ((), ())),
                          preferred_element_type=jnp.float32)
    cnt = jnp.sum(oh, axis=0, keepdims=True)
    pooled = seg / jnp.clip(cnt, 1.0, None).reshape(G, 1)
    out_ref[...] = (jnp.dot(pooled, wl_ref[...],
                            preferred_element_type=jnp.float32) + bl_ref[...])


def kernel(x, edge_index, batch, W1, b1, g1, be1, Wc, bc, gc, bec, Wl, bl):
    loop = jnp.arange(N, dtype=jnp.int32)
    src = jnp.concatenate([edge_index[0].astype(jnp.int32), loop,
                           jnp.zeros((E_PAD - E_TOT,), jnp.int32)])
    dst = jnp.concatenate([edge_index[1].astype(jnp.int32), loop,
                           jnp.full((E_PAD - E_TOT,), N, jnp.int32)])
    x_pad = jnp.zeros((NPAD, D), jnp.float32).at[:N].set(x)
    batch_pad = jnp.concatenate([batch.astype(jnp.int32),
                                 jnp.full((NPAD - N,), G, jnp.int32)])
    z_rows = jnp.zeros((ROWS, H), jnp.float32)
    ones_rows = jnp.ones((CHUNK, H), jnp.float32)

    deg16 = _sc_degree(dst, ones_rows, z_rows)

    y, dinv = pl.pallas_call(
        _tc_prep,
        out_shape=(jax.ShapeDtypeStruct((NPAD, H), jnp.float32),
                   jax.ShapeDtypeStruct((NPAD, 1), jnp.float32)),
    )(x_pad, W1, deg16)

    biases = [b1, bc[0], bc[1]]
    gammas = [g1, gc[0], gc[1]]
    betas = [be1, bec[0], bec[1]]

    for layer in range(3):
        p = _sc_gather_scatter(y, src, dst, z_rows)
        b_ = biases[layer].reshape(1, H)
        g_ = gammas[layer].reshape(1, H)
        be_ = betas[layer].reshape(1, H)
        if layer < 2:
            y = pl.pallas_call(
                _tc_mid,
                out_shape=jax.ShapeDtypeStruct((NPAD, H), jnp.float32),
            )(p, dinv, b_, g_, be_, Wc[layer])
        else:
            out = pl.pallas_call(
                _tc_final,
                out_shape=jax.ShapeDtypeStruct((G, C), jnp.float32),
            )(p, dinv, b_, g_, be_, batch_pad, Wl, bl.reshape(1, C))
    return out


# speedup vs baseline: 17.0633x; 1.5565x over previous
import functools

import jax
import jax.numpy as jnp
from jax import lax
from jax.experimental import pallas as pl
from jax.experimental.pallas import tpu as pltpu
from jax.experimental.pallas import tpu_sc as plsc

N = 10000
NPAD = 10240
D = 128
H = 128
C = 40
G = 64
E_TOT = 320000 + N
CHUNK = 128
NWORK = 32
NBUF = 6
NCH = 84
E_PAD = NCH * CHUNK * NWORK
ROWS = NPAD // 16

_mesh = plsc.VectorSubcoreMesh(core_axis_name="c", subcore_axis_name="s")


@functools.partial(
    pl.kernel,
    out_type=jax.ShapeDtypeStruct((2, NPAD, H), jnp.float32),
    mesh=_mesh,
    scratch_types=[
        pltpu.VMEM((NCH, CHUNK), jnp.int32),
        pltpu.VMEM((CHUNK, H), jnp.float32),
        pltpu.VMEM_SHARED((NPAD, H), jnp.float32),
        pltpu.SemaphoreType.DMA,
    ],
)
def _sc_degree(dst_hbm, ones_hbm, z16_hbm, out_hbm, dst_v, ones_v, acc, sem):
    c = lax.axis_index("c")
    s = lax.axis_index("s")
    w = c * 16 + s
    pltpu.sync_copy(z16_hbm, acc.at[pl.ds(s * ROWS, ROWS)])
    pltpu.sync_copy(ones_hbm, ones_v)
    pltpu.sync_copy(dst_hbm.at[w], dst_v)
    plsc.subcore_barrier()

    def body(g, carry):
        for b in range(NBUF):
            pltpu.async_copy(ones_v, acc.at[dst_v.at[g * NBUF + b]], sem,
                             add=True)
        for b in range(NBUF):
            pltpu.make_async_copy(ones_v, acc.at[dst_v.at[g * NBUF + b]],
                                  sem).wait()
        return carry

    lax.fori_loop(0, NCH // NBUF, body, 0)
    plsc.subcore_barrier()
    pltpu.sync_copy(acc.at[pl.ds(s * ROWS, ROWS)],
                    out_hbm.at[c, pl.ds(s * ROWS, ROWS)])


@functools.partial(
    pl.kernel,
    out_type=jax.ShapeDtypeStruct((2, NPAD, H), jnp.float32),
    mesh=_mesh,
    scratch_types=[
        pltpu.VMEM((NCH, CHUNK), jnp.int32),
        pltpu.VMEM((NCH, CHUNK), jnp.int32),
        pltpu.VMEM((CHUNK, H), jnp.float32),
        pltpu.VMEM_SHARED((NPAD, H), jnp.float32),
        pltpu.SemaphoreType.DMA,
    ],
)
def _sc_gather_scatter(y_hbm, src_hbm, dst_hbm, z_hbm, out_hbm,
                       src_v, dst_v, rows_v, acc, sem):
    c = lax.axis_index("c")
    s = lax.axis_index("s")
    w = c * 16 + s
    pltpu.sync_copy(z_hbm, acc.at[pl.ds(s * ROWS, ROWS)])
    pltpu.sync_copy(src_hbm.at[w], src_v)
    pltpu.sync_copy(dst_hbm.at[w], dst_v)
    plsc.subcore_barrier()

    pltpu.async_copy(y_hbm.at[src_v.at[0]], rows_v, sem)

    def body(g, carry):
        pltpu.make_async_copy(y_hbm.at[src_v.at[g]], rows_v, sem).wait()
        pltpu.sync_copy(rows_v, acc.at[dst_v.at[g]], add=True)
        pltpu.async_copy(y_hbm.at[src_v.at[g + 1]], rows_v, sem)
        return carry

    lax.fori_loop(0, NCH - 1, body, 0)
    pltpu.make_async_copy(y_hbm.at[src_v.at[NCH - 1]], rows_v, sem).wait()
    pltpu.sync_copy(rows_v, acc.at[dst_v.at[NCH - 1]], add=True)
    plsc.subcore_barrier()
    pltpu.sync_copy(acc.at[pl.ds(s * ROWS, ROWS)],
                    out_hbm.at[c, pl.ds(s * ROWS, ROWS)])


def _valid_mask():
    return lax.broadcasted_iota(jnp.int32, (NPAD, 1), 0) < N


def _tc_prep(x_ref, w1_ref, deg_ref, y_ref, dinv_ref):
    deg16 = deg_ref[0] + deg_ref[1]
    deg = deg16[:, 0:1]
    dinv = jnp.where(deg > 0.0, lax.rsqrt(deg), 0.0)
    xw = jnp.dot(x_ref[...], w1_ref[...], preferred_element_type=jnp.float32)
    y_ref[...] = xw * dinv
    dinv_ref[...] = dinv


def _bn_relu(p0, p1, dinv, b, g, be):
    z = (p0 + p1) * dinv + b
    mask = _valid_mask()
    zm = jnp.where(mask, z, 0.0)
    m = jnp.sum(zm, axis=0, keepdims=True) * (1.0 / N)
    dz = jnp.where(mask, z - m, 0.0)
    v = jnp.sum(dz * dz, axis=0, keepdims=True) * (1.0 / N)
    return jnp.maximum(g * (z - m) * lax.rsqrt(v + 1e-5) + be, 0.0)


def _tc_mid(p_ref, dinv_ref, b_ref, g_ref, be_ref, wn_ref, y_ref):
    dinv = dinv_ref[...]
    h = _bn_relu(p_ref[0], p_ref[1], dinv, b_ref[...], g_ref[...], be_ref[...])
    y_ref[...] = jnp.dot(h, wn_ref[...], preferred_element_type=jnp.float32) * dinv


def _tc_final(p_ref, dinv_ref, b_ref, g_ref, be_ref, batch_ref, wl_ref,
              bl_ref, out_ref):
    h = _bn_relu(p_ref[0], p_ref[1], dinv_ref[...], b_ref[...], g_ref[...],
                 be_ref[...])
    h = jnp.where(_valid_mask(), h, 0.0)
    gids = lax.broadcasted_iota(jnp.int32, (NPAD, G), 1)
    oh = (batch_ref[...].reshape(NPAD, 1) == gids).astype(jnp.float32)
    seg = lax.dot_general(oh, h, (((0,), (0,)), ((), ())),
                          preferred_element_type=jnp.float32)
    cnt = jnp.sum(oh, axis=0, keepdims=True)
    pooled = seg / jnp.clip(cnt, 1.0, None).reshape(G, 1)
    out_ref[...] = (jnp.dot(pooled, wl_ref[...],
                            preferred_element_type=jnp.float32) + bl_ref[...])


def kernel(x, edge_index, batch, W1, b1, g1, be1, Wc, bc, gc, bec, Wl, bl):
    loop = jnp.arange(N, dtype=jnp.int32)
    npad_e = E_PAD - E_TOT
    pad_spread = jnp.arange(npad_e, dtype=jnp.int32) % (NPAD - N)
    src = jnp.concatenate([edge_index[0].astype(jnp.int32), loop,
                           pad_spread]).reshape(NWORK, NCH, CHUNK)
    dst = jnp.concatenate([edge_index[1].astype(jnp.int32), loop,
                           N + pad_spread]).reshape(NWORK, NCH, CHUNK)
    x_pad = jnp.zeros((NPAD, D), jnp.float32).at[:N].set(x)
    batch_pad = jnp.concatenate([batch.astype(jnp.int32),
                                 jnp.full((NPAD - N,), G, jnp.int32)])
    z_rows = jnp.zeros((ROWS, H), jnp.float32)
    ones_rows = jnp.ones((CHUNK, H), jnp.float32)

    deg16 = _sc_degree(dst, ones_rows, z_rows)

    y, dinv = pl.pallas_call(
        _tc_prep,
        out_shape=(jax.ShapeDtypeStruct((NPAD, H), jnp.float32),
                   jax.ShapeDtypeStruct((NPAD, 1), jnp.float32)),
    )(x_pad, W1, deg16)

    biases = [b1, bc[0], bc[1]]
    gammas = [g1, gc[0], gc[1]]
    betas = [be1, bec[0], bec[1]]

    for layer in range(3):
        p = _sc_gather_scatter(y, src, dst, z_rows)
        b_ = biases[layer].reshape(1, H)
        g_ = gammas[layer].reshape(1, H)
        be_ = betas[layer].reshape(1, H)
        if layer < 2:
            y = pl.pallas_call(
                _tc_mid,
                out_shape=jax.ShapeDtypeStruct((NPAD, H), jnp.float32),
            )(p, dinv, b_, g_, be_, Wc[layer])
        else:
            out = pl.pallas_call(
                _tc_final,
                out_shape=jax.ShapeDtypeStruct((G, C), jnp.float32),
            )(p, dinv, b_, g_, be_, batch_pad, Wl, bl.reshape(1, C))
    return out
